# Initial kernel scaffold; baseline (speedup 1.0000x reference)
import jax
import jax.numpy as jnp
from jax.experimental import pallas as pl


def kernel(x, edge_index, W1, b1, W2, b2):
    raise NotImplementedError("write your pallas kernel here")


# speedup vs baseline: 15.2096x; 15.2096x over previous
import functools

import jax
import jax.numpy as jnp
from jax import lax
from jax.experimental import pallas as pl
from jax.experimental.pallas import tpu as pltpu
from jax.experimental.pallas import tpu_sc as plsc

N = 10000
NPAD = 10240
E = 320000
NC = 2
NS = 16
NW = NC * NS
C = 128
CHUNKS = E // C
ITERS = -(-CHUNKS // NW)
ROWS_PER_SUB = NPAD // NS

_mesh = plsc.VectorSubcoreMesh(core_axis_name="c", subcore_axis_name="s")


def _make_edge_scatter(D):
  @functools.partial(
      pl.kernel,
      mesh=_mesh,
      out_type=jax.ShapeDtypeStruct((NC, NPAD, D), jnp.float32),
      scratch_types=[
          pltpu.VMEM((C,), jnp.int32),
          pltpu.VMEM((C,), jnp.int32),
          pltpu.VMEM((C, D), jnp.float32),
          pltpu.VMEM_SHARED((NPAD, D), jnp.float32),
          pltpu.SemaphoreType.DMA,
      ],
  )
  def scat(g_hbm, src_hbm, dst_hbm, zeros_hbm, out_hbm,
           src_v, dst_v, rows_v, acc, sem):
    cid = lax.axis_index("c")
    sid = lax.axis_index("s")
    wid = sid * NC + cid
    base = sid * ROWS_PER_SUB
    pltpu.sync_copy(zeros_hbm.at[pl.ds(base, ROWS_PER_SUB)],
                    acc.at[pl.ds(base, ROWS_PER_SUB)])
    plsc.subcore_barrier()

    def body(t, carry):
      k = wid + t * NW

      @pl.when(k < CHUNKS)
      def _():
        off = k * C
        pltpu.sync_copy(src_hbm.at[pl.ds(off, C)], src_v)
        pltpu.sync_copy(dst_hbm.at[pl.ds(off, C)], dst_v)
        pltpu.async_copy(g_hbm.at[src_v], rows_v, sem).wait()
        pltpu.sync_copy(rows_v, acc.at[dst_v], add=True)

      return carry

    lax.fori_loop(0, ITERS, body, 0)
    plsc.subcore_barrier()
    pltpu.sync_copy(acc.at[pl.ds(base, ROWS_PER_SUB)],
                    out_hbm.at[cid].at[pl.ds(base, ROWS_PER_SUB)])

  return scat


_scatter128 = _make_edge_scatter(128)

DEGW = 128


@functools.partial(
    pl.kernel,
    mesh=_mesh,
    out_type=jax.ShapeDtypeStruct((NC, NPAD, DEGW), jnp.float32),
    scratch_types=[
        pltpu.VMEM((C,), jnp.int32),
        pltpu.VMEM((C, DEGW), jnp.float32),
        pltpu.VMEM_SHARED((NPAD, DEGW), jnp.float32),
    ],
)
def _deg_kernel(dst_hbm, zeros_hbm, ones_hbm, out_hbm, dst_v, ones_v, acc):
  cid = lax.axis_index("c")
  sid = lax.axis_index("s")
  wid = sid * NC + cid
  base = sid * ROWS_PER_SUB
  pltpu.sync_copy(ones_hbm, ones_v)
  pltpu.sync_copy(zeros_hbm.at[pl.ds(base, ROWS_PER_SUB)],
                  acc.at[pl.ds(base, ROWS_PER_SUB)])
  plsc.subcore_barrier()

  def body(t, carry):
    k = wid + t * NW

    @pl.when(k < CHUNKS)
    def _():
      off = k * C
      pltpu.sync_copy(dst_hbm.at[pl.ds(off, C)], dst_v)
      pltpu.sync_copy(ones_v, acc.at[dst_v], add=True)

    return carry

  lax.fori_loop(0, ITERS, body, 0)
  plsc.subcore_barrier()
  pltpu.sync_copy(acc.at[pl.ds(base, ROWS_PER_SUB)],
                  out_hbm.at[cid].at[pl.ds(base, ROWS_PER_SUB)])


R = 1000
GRID = N // R


def _dinv_of(degp_ref):
  deg = degp_ref[0, :, 0] + degp_ref[1, :, 0] + 1.0
  return lax.rsqrt(deg)


def _mm1_body(x_ref, w_ref, degp_ref, o_ref):
  dinv = _dinv_of(degp_ref)
  h = jnp.dot(x_ref[...], w_ref[...], preferred_element_type=jnp.float32)
  o_ref[...] = h * dinv[:, None]


def _mm2_body(s_ref, g1_ref, degp_ref, b1_ref, w2_ref, o_ref):
  dinv = _dinv_of(degp_ref)
  a = (s_ref[0] + s_ref[1] + g1_ref[...]) * dinv[:, None] + b1_ref[...]
  a = jnp.maximum(a, 0.0)
  h = jnp.dot(a, w2_ref[...], preferred_element_type=jnp.float32)
  o_ref[...] = jnp.concatenate(
      [h * dinv[:, None], jnp.zeros((R, 64), jnp.float32)], axis=1)


def _fin_body(s_ref, g2_ref, degp_ref, b2_ref, o_ref):
  dinv = _dinv_of(degp_ref)
  z = ((s_ref[0, :, :64] + s_ref[1, :, :64] + g2_ref[:, :64]) * dinv[:, None]
       + b2_ref[...])
  m = jnp.max(z, axis=1, keepdims=True)
  zs = z - m
  o_ref[...] = zs - jnp.log(jnp.sum(jnp.exp(zs), axis=1, keepdims=True))


def _row_spec(width):
  return pl.BlockSpec((R, width), lambda i: (i, 0))


def _pair_spec(width):
  return pl.BlockSpec((NC, R, width), lambda i: (0, i, 0))


_degp_spec = pl.BlockSpec((NC, R, DEGW), lambda i: (0, i, 0))
_full = lambda shape: pl.BlockSpec(shape, lambda i: (0,) * len(shape))


def _mm1_call(x, W1, degp):
  return pl.pallas_call(
      _mm1_body,
      grid=(GRID,),
      in_specs=[_row_spec(128), _full((128, 128)), _degp_spec],
      out_specs=_row_spec(128),
      out_shape=jax.ShapeDtypeStruct((N, 128), jnp.float32),
  )(x, W1, degp)


def _mm2_call(s1, g1, degp, b1, W2):
  return pl.pallas_call(
      _mm2_body,
      grid=(GRID,),
      in_specs=[_pair_spec(128), _row_spec(128), _degp_spec,
                _full((1, 128)), _full((128, 64))],
      out_specs=_row_spec(128),
      out_shape=jax.ShapeDtypeStruct((N, 128), jnp.float32),
  )(s1, g1, degp, b1, W2)


def _fin_call(s2, g2, degp, b2):
  return pl.pallas_call(
      _fin_body,
      grid=(GRID,),
      in_specs=[_pair_spec(128), _row_spec(128), _degp_spec, _full((1, 64))],
      out_specs=_row_spec(64),
      out_shape=jax.ShapeDtypeStruct((N, 64), jnp.float32),
  )(s2, g2, degp, b2)


@jax.jit
def kernel(x, edge_index, W1, b1, W2, b2):
  ei = edge_index.astype(jnp.int32)
  src, dst = ei[0], ei[1]
  zeros128 = jnp.zeros((NPAD, 128), jnp.float32)
  zerosw = jnp.zeros((NPAD, DEGW), jnp.float32)
  ones = jnp.ones((C, DEGW), jnp.float32)

  degp = _deg_kernel(dst, zerosw, ones)
  g1 = _mm1_call(x, W1, degp)
  s1 = _scatter128(g1, src, dst, zeros128)
  g2 = _mm2_call(s1, g1, degp, b1.reshape(1, -1), W2)
  s2 = _scatter128(g2, src, dst, zeros128)
  return _fin_call(s2, g2, degp, b2.reshape(1, -1))
